# baseline (speedup 1.0000x reference)
import functools

import jax
import jax.numpy as jnp
from jax import lax
from jax.experimental import pallas as pl
from jax.experimental.pallas import tpu as pltpu
from jax.experimental.pallas import tpu_sc as plsc

N = 100000
D = 128
G = 64
L = 16
NC = 2
NS = 16
NW = NC * NS
SUB = 16
NPAD = 100096
NSUB = NPAD // SUB
CHUNK = 256
SEGS_PER_W = G // NW
M = 50400
R = 800

_mesh = plsc.VectorSubcoreMesh(core_axis_name="c", subcore_axis_name="s")


def _pool_body(x_hbm, batch_hbm, bsub_hbm, out_ms, out_cnt,
               bsub_v, win_v, buf0, buf1, stage, stagec, semw, sem0, sem1):
    w = lax.axis_index("c") * NS + lax.axis_index("s")
    iota = lax.iota(jnp.int32, L)

    pltpu.sync_copy(bsub_hbm, bsub_v)
    svec = jnp.minimum(2 * w + iota, G)
    lo = jnp.zeros((L,), jnp.int32)
    hi = jnp.full((L,), NSUB, jnp.int32)
    for _ in range(13):
        active = lo < hi
        mid = (lo + hi) // 2
        vals = plsc.load_gather(bsub_v, [jnp.minimum(mid, NSUB - 1)])
        less = vals < svec
        lo = jnp.where(jnp.logical_and(active, less), mid + 1, lo)
        hi = jnp.where(jnp.logical_and(active, jnp.logical_not(less)), mid, hi)

    wbs = []
    for j in range(SEGS_PER_W + 1):
        p = jnp.sum(jnp.where(iota == j, lo, 0))
        wb = SUB * jnp.maximum(p - 1, 0)
        pltpu.async_copy(batch_hbm.at[pl.ds(pl.multiple_of(wb, 8), SUB)],
                         win_v.at[j], semw)
        wbs.append(wb)
    bounds = []
    for j in range(SEGS_PER_W + 1):
        pltpu.make_async_copy(batch_hbm.at[pl.ds(0, SUB)], win_v.at[j],
                              semw).wait()
    for j in range(SEGS_PER_W + 1):
        s_j = 2 * w + j
        in_win = jnp.sum(jnp.where(win_v[j] < s_j, 1, 0))
        bounds.append(wbs[j] + in_win)

    for j in range(SEGS_PER_W):
        seg = w * SEGS_PER_W + j
        row_lo = jnp.minimum(bounds[j], M)
        row_hi = jnp.minimum(bounds[j + 1], M)
        nrows = row_hi - row_lo
        w0 = (row_lo // 8) * 8
        nch = jnp.where(nrows > 0, (row_hi - w0 + CHUNK - 1) // CHUNK, 0)

        def dma_slice(c):
            return x_hbm.at[
                pl.ds(pl.multiple_of(jnp.minimum(w0 + c * CHUNK, N - CHUNK), 8),
                      CHUNK)]

        def start_copy(c, buf, sem):
            pltpu.async_copy(dma_slice(c), buf, sem)

        def wait_copy(c, buf, sem):
            pltpu.make_async_copy(dma_slice(c), buf, sem).wait()

        def process(c, buf, carry):
            wbase = w0 + c * CHUNK
            dma_base = pl.multiple_of(jnp.minimum(wbase, N - CHUNK), 8)
            shift = wbase - dma_base
            r0 = jnp.maximum(row_lo - wbase, 0)
            r1 = jnp.minimum(row_hi - wbase, CHUNK)

            def accum(rr, c2):
                vs = [buf[rr, pl.ds(k * L, L)] for k in range(D // L)]
                mx = tuple(jnp.maximum(c2[k], vs[k]) for k in range(D // L))
                sm = tuple(c2[D // L + k] + vs[k] for k in range(D // L))
                return mx + sm

            def pair_rows(i, c2):
                rr = shift + r0 + 2 * i
                return accum(rr + 1, accum(rr, c2))

            nr = r1 - r0
            carry = lax.fori_loop(0, nr // 2, pair_rows, carry)
            return lax.cond(
                nr % 2 == 1,
                lambda c2: accum(shift + r1 - 1, c2),
                lambda c2: c2, carry)

        @pl.when(nch > 0)
        def _():
            start_copy(0, buf0, sem0)

        def pair_body(jp, carry):
            c0 = 2 * jp
            c1 = c0 + 1

            def with_c1(cr):
                start_copy(c1, buf1, sem1)
                return cr

            carry = lax.cond(c1 < nch, with_c1, lambda cr: cr, carry)
            wait_copy(c0, buf0, sem0)
            carry = process(c0, buf0, carry)

            def with_c1_tail(cr):
                def issue_next(cr2):
                    start_copy(c1 + 1, buf0, sem0)
                    return cr2

                cr = lax.cond(c1 + 1 < nch, issue_next, lambda cr2: cr2, cr)
                wait_copy(c1, buf1, sem1)
                return process(c1, buf1, cr)

            return lax.cond(c1 < nch, with_c1_tail, lambda cr: cr, carry)

        init = tuple(jnp.full((L,), -jnp.inf, jnp.float32) for _ in range(D // L)) \
            + tuple(jnp.zeros((L,), jnp.float32) for _ in range(D // L))
        res = lax.fori_loop(0, (nch + 1) // 2, pair_body, init)

        for k in range(D // L):
            stage[pl.ds(k * L, L)] = res[k]
            stage[pl.ds(D + k * L, L)] = res[D // L + k]
        stagec[...] = jnp.broadcast_to(nrows.astype(jnp.float32), (L,))
        pltpu.sync_copy(stage, out_ms.at[seg])
        pltpu.sync_copy(stagec, out_cnt.at[seg])


_pool_kernel = functools.partial(
    pl.kernel,
    out_type=[
        jax.ShapeDtypeStruct((G, 2 * D), jnp.float32),
        jax.ShapeDtypeStruct((G, L), jnp.float32),
    ],
    mesh=_mesh,
    compiler_params=pltpu.CompilerParams(needs_layout_passes=False),
    scratch_types=[
        pltpu.VMEM((NSUB,), jnp.int32),
        pltpu.VMEM((SEGS_PER_W + 1, SUB), jnp.int32),
        pltpu.VMEM((CHUNK, D), jnp.float32),
        pltpu.VMEM((CHUNK, D), jnp.float32),
        pltpu.VMEM((2 * D,), jnp.float32),
        pltpu.VMEM((L,), jnp.float32),
        pltpu.SemaphoreType.DMA,
        pltpu.SemaphoreType.DMA,
        pltpu.SemaphoreType.DMA,
    ],
)(_pool_body)


def _tc_body(x_ref, b_ref, omax, osum, ocnt):
    i = pl.program_id(0)

    @pl.when(i == 0)
    def _():
        omax[...] = jnp.full((G, D), -jnp.inf, jnp.float32)
        osum[...] = jnp.zeros((G, D), jnp.float32)
        ocnt[...] = jnp.zeros((G, D), jnp.float32)

    xb = x_ref[...]
    bb = b_ref[...]
    iot = lax.broadcasted_iota(jnp.int32, (R, G), 1)
    oh = jnp.where(bb == iot, 1.0, 0.0)
    osum[...] += lax.dot_general(oh, xb, (((0,), (0,)), ((), ())),
                                 preferred_element_type=jnp.float32)
    ocnt[...] += jnp.broadcast_to(jnp.sum(oh, axis=0)[:, None], (G, D))

    seg_lo = bb[0, 0]
    seg_hi = bb[R - 1, 0]

    def sbody(s, carry):
        m = jnp.max(jnp.where(bb == s, xb, -jnp.inf), axis=0)
        omax[pl.ds(s, 1), :] = jnp.maximum(omax[pl.ds(s, 1), :], m[None, :])
        return carry

    lax.fori_loop(seg_lo, seg_hi + 1, sbody, 0)


_tc_kernel = pl.pallas_call(
    _tc_body,
    grid=((N - M) // R,),
    in_specs=[
        pl.BlockSpec((R, D), lambda i: (i + M // R, 0)),
        pl.BlockSpec((R, 1), lambda i: (i + M // R, 0)),
    ],
    out_specs=[
        pl.BlockSpec((G, D), lambda i: (0, 0)),
        pl.BlockSpec((G, D), lambda i: (0, 0)),
        pl.BlockSpec((G, D), lambda i: (0, 0)),
    ],
    out_shape=[
        jax.ShapeDtypeStruct((G, D), jnp.float32),
        jax.ShapeDtypeStruct((G, D), jnp.float32),
        jax.ShapeDtypeStruct((G, D), jnp.float32),
    ],
)


def _merge_body(scms_ref, sccnt_ref, tmax_ref, tsum_ref, tcnt_ref, o_ref):
    mx = jnp.maximum(scms_ref[:, :D], tmax_ref[...])
    sm = scms_ref[:, D:] + tsum_ref[...]
    cnt = sccnt_ref[:, 0:1] + tcnt_ref[:, 0:1]
    mean = sm / jnp.maximum(cnt, 1.0)
    o_ref[...] = jnp.concatenate([mx, mean], axis=1)


_merge_kernel = pl.pallas_call(
    _merge_body,
    out_shape=jax.ShapeDtypeStruct((G, 2 * D), jnp.float32),
)


def kernel(x, batch):
    batch = batch.astype(jnp.int32)
    bsub = jnp.concatenate(
        [batch[::SUB], jnp.full((NSUB - N // SUB,), jnp.int32(2**30))])
    scms, sccnt = _pool_kernel(x, batch, bsub)
    tmax, tsum, tcnt = _tc_kernel(x, batch.reshape(N, 1))
    return _merge_kernel(scms, sccnt, tmax, tsum, tcnt)


# speedup vs baseline: 2.6690x; 2.6690x over previous
import functools

import jax
import jax.numpy as jnp
from jax import lax
from jax.experimental import pallas as pl
from jax.experimental.pallas import tpu as pltpu
from jax.experimental.pallas import tpu_sc as plsc

N = 100000
D = 128
G = 64
L = 16
NC = 2
NS = 16
NW = NC * NS
SUB = 16
NPAD = 100096
NSUB = NPAD // SUB
CHUNK = 128
NBUF = 4
SEGS_PER_W = G // NW

_mesh = plsc.VectorSubcoreMesh(core_axis_name="c", subcore_axis_name="s")


def _pool_body(x_hbm, batch_hbm, bsub_hbm, out_hbm,
               bsub_v, win_v, buf0, buf1, buf2, buf3, stage,
               semw, sem0, sem1, sem2, sem3):
    bufs = (buf0, buf1, buf2, buf3)
    sems = (sem0, sem1, sem2, sem3)
    w = lax.axis_index("c") * NS + lax.axis_index("s")
    iota = lax.iota(jnp.int32, L)

    pltpu.sync_copy(bsub_hbm, bsub_v)
    svec = jnp.minimum(2 * w + iota, G)
    lo = jnp.zeros((L,), jnp.int32)
    hi = jnp.full((L,), NSUB, jnp.int32)
    for _ in range(13):
        active = lo < hi
        mid = (lo + hi) // 2
        vals = plsc.load_gather(bsub_v, [jnp.minimum(mid, NSUB - 1)])
        less = vals < svec
        lo = jnp.where(jnp.logical_and(active, less), mid + 1, lo)
        hi = jnp.where(jnp.logical_and(active, jnp.logical_not(less)), mid, hi)

    wbs = []
    for j in range(SEGS_PER_W + 1):
        p = jnp.sum(jnp.where(iota == j, lo, 0))
        wb = SUB * jnp.maximum(p - 1, 0)
        pltpu.async_copy(batch_hbm.at[pl.ds(pl.multiple_of(wb, 8), SUB)],
                         win_v.at[j], semw)
        wbs.append(wb)
    bounds = []
    for j in range(SEGS_PER_W + 1):
        pltpu.make_async_copy(batch_hbm.at[pl.ds(0, SUB)], win_v.at[j],
                              semw).wait()
    for j in range(SEGS_PER_W + 1):
        s_j = 2 * w + j
        in_win = jnp.sum(jnp.where(win_v[j] < s_j, 1, 0))
        bounds.append(wbs[j] + in_win)

    for j in range(SEGS_PER_W):
        seg = w * SEGS_PER_W + j
        row_lo = bounds[j]
        row_hi = bounds[j + 1]
        nrows = row_hi - row_lo
        w0 = (row_lo // 8) * 8
        nch = jnp.where(nrows > 0, (row_hi - w0 + CHUNK - 1) // CHUNK, 0)

        def dma_slice(c):
            return x_hbm.at[
                pl.ds(pl.multiple_of(jnp.minimum(w0 + c * CHUNK, N - CHUNK), 8),
                      CHUNK)]

        def start_copy(c, buf, sem):
            pltpu.async_copy(dma_slice(c), buf, sem)

        def wait_copy(c, buf, sem):
            pltpu.make_async_copy(dma_slice(c), buf, sem).wait()

        def process(c, buf, carry):
            wbase = w0 + c * CHUNK
            dma_base = pl.multiple_of(jnp.minimum(wbase, N - CHUNK), 8)
            shift = wbase - dma_base
            r0 = jnp.maximum(row_lo - wbase, 0)
            r1 = jnp.minimum(row_hi - wbase, CHUNK)

            def accum(rr, c2):
                vs = [buf[rr, pl.ds(k * L, L)] for k in range(D // L)]
                mx = tuple(jnp.maximum(c2[k], vs[k]) for k in range(D // L))
                sm = tuple(c2[D // L + k] + vs[k] for k in range(D // L))
                return mx + sm

            def pair_rows(i, c2):
                rr = shift + r0 + 2 * i
                return accum(rr + 1, accum(rr, c2))

            nr = r1 - r0
            carry = lax.fori_loop(0, nr // 2, pair_rows, carry)
            return lax.cond(
                nr % 2 == 1,
                lambda c2: accum(shift + r1 - 1, c2),
                lambda c2: c2, carry)

        for u in range(NBUF - 1):
            @pl.when(u < nch)
            def _(u=u):
                start_copy(u, bufs[u], sems[u])

        def ring_body(jq, carry):
            for u in range(NBUF):
                c = NBUF * jq + u

                def do(cr, c=c, u=u):
                    def issue(cr2, c=c, u=u):
                        v = (u + NBUF - 1) % NBUF
                        start_copy(c + NBUF - 1, bufs[v], sems[v])
                        return cr2

                    cr = lax.cond(c + NBUF - 1 < nch, issue,
                                  lambda cr2: cr2, cr)
                    wait_copy(c, bufs[u], sems[u])
                    return process(c, bufs[u], cr)

                carry = lax.cond(c < nch, do, lambda cr: cr, carry)
            return carry

        init = tuple(jnp.full((L,), -jnp.inf, jnp.float32) for _ in range(D // L)) \
            + tuple(jnp.zeros((L,), jnp.float32) for _ in range(D // L))
        res = lax.fori_loop(0, (nch + NBUF - 1) // NBUF, ring_body, init)

        cnt_vec = jnp.broadcast_to(
            jnp.maximum(nrows, 1).astype(jnp.float32), (L,))
        inv = jnp.ones((L,), jnp.float32) / cnt_vec
        for k in range(D // L):
            stage[pl.ds(k * L, L)] = res[k]
            stage[pl.ds(D + k * L, L)] = res[D // L + k] * inv
        pltpu.sync_copy(stage, out_hbm.at[seg])


_pool_kernel = functools.partial(
    pl.kernel,
    out_type=jax.ShapeDtypeStruct((G, 2 * D), jnp.float32),
    mesh=_mesh,
    compiler_params=pltpu.CompilerParams(needs_layout_passes=False),
    scratch_types=[
        pltpu.VMEM((NSUB,), jnp.int32),
        pltpu.VMEM((SEGS_PER_W + 1, SUB), jnp.int32),
        pltpu.VMEM((CHUNK, D), jnp.float32),
        pltpu.VMEM((CHUNK, D), jnp.float32),
        pltpu.VMEM((CHUNK, D), jnp.float32),
        pltpu.VMEM((CHUNK, D), jnp.float32),
        pltpu.VMEM((2 * D,), jnp.float32),
        pltpu.SemaphoreType.DMA,
        pltpu.SemaphoreType.DMA,
        pltpu.SemaphoreType.DMA,
        pltpu.SemaphoreType.DMA,
        pltpu.SemaphoreType.DMA,
    ],
)(_pool_body)


def kernel(x, batch):
    batch = batch.astype(jnp.int32)
    bsub = jnp.concatenate(
        [batch[::SUB], jnp.full((NSUB - N // SUB,), jnp.int32(2**30))])
    return _pool_kernel(x, batch, bsub)
